# baseline (speedup 1.0000x reference)
import functools

import jax
import jax.numpy as jnp
import numpy as np
from jax import lax
from jax.experimental import pallas as pl
from jax.experimental.pallas import tpu as pltpu
from jax.experimental.pallas import tpu_sc as plsc

NUM_EMB = 1000000
DIM = 64
SEQ = 16384
POS = 50
NW = 32
BLK = 128
IC = SEQ // BLK
N_BLOCKS = POS * IC
BPW = N_BLOCKS // NW

def _body(ids_hbm, table_hbm, out_hbm, idx0, idx1, rows0, rows1, tbuf0, tbuf1,
          gsem0, gsem1, wsem0, wsem1):
    nc = 2
    wid = lax.axis_index("s") * nc + lax.axis_index("c")
    lanes = lax.iota(jnp.int32, 16)
    dmod = [(lanes + s) & 15 for s in range(16)]
    zeros = lanes & 0

    def fire(b, idx_v, rows_v, gsem):
        k = b // IC
        ic = b % IC
        pltpu.sync_copy(ids_hbm.at[k, pl.ds(ic * BLK, BLK)], idx_v)
        pltpu.async_copy(table_hbm.at[idx_v], rows_v, gsem)

    def drain_gather(rows_v, gsem):
        pltpu.make_async_copy(table_hbm.at[idx0], rows_v, gsem).wait()

    def transpose(rows_v, tbuf):
        def cg_step(cg, carry):
            toks = lanes + cg * 16
            for j0 in range(0, DIM, 16):
                for s in range(16):
                    jf = dmod[s] + j0
                    vals = plsc.load_gather(rows_v, [toks, jf])
                    plsc.store_scatter(tbuf, [jf >> 3, zeros, jf & 7, toks],
                                       vals)
            return carry
        lax.fori_loop(0, BLK // 16, cg_step, 0)

    def write(b, tbuf, wsem):
        k = b // IC
        ic = b % IC
        pltpu.async_copy(tbuf, out_hbm.at[k, :, pl.ds(ic, 1), :, :], wsem)

    def drain_write(tbuf, wsem):
        pltpu.make_async_copy(tbuf, out_hbm.at[0, :, pl.ds(0, 1), :, :],
                              wsem).wait()

    base = wid * BPW
    fire(base, idx0, rows0, gsem0)

    def step(s, carry):
        b0 = base + 2 * s

        fire(b0 + 1, idx1, rows1, gsem1)
        drain_gather(rows0, gsem0)

        @pl.when(s > 0)
        def _():
            drain_write(tbuf0, wsem0)
        transpose(rows0, tbuf0)
        write(b0, tbuf0, wsem0)

        @pl.when(s < BPW // 2 - 1)
        def _():
            fire(b0 + 2, idx0, rows0, gsem0)
        drain_gather(rows1, gsem1)

        @pl.when(s > 0)
        def _():
            drain_write(tbuf1, wsem1)
        transpose(rows1, tbuf1)
        write(b0 + 1, tbuf1, wsem1)
        return carry

    lax.fori_loop(0, BPW // 2, step, 0)
    drain_write(tbuf0, wsem0)
    drain_write(tbuf1, wsem1)


def kernel(token_ids, embedding_weights):
    ids_t = token_ids.T.astype(jnp.int32)
    mesh = plsc.VectorSubcoreMesh(core_axis_name="c", subcore_axis_name="s")
    k = functools.partial(
        pl.kernel,
        mesh=mesh,
        out_type=jax.ShapeDtypeStruct((POS, 8, SEQ // 128, 8, 128),
                                      jnp.float32),
        scratch_types=[
            pltpu.VMEM((BLK,), jnp.int32),
            pltpu.VMEM((BLK,), jnp.int32),
            pltpu.VMEM((BLK, DIM), jnp.float32),
            pltpu.VMEM((BLK, DIM), jnp.float32),
            pltpu.VMEM((8, 1, 8, 128), jnp.float32),
            pltpu.VMEM((8, 1, 8, 128), jnp.float32),
            pltpu.SemaphoreType.DMA,
            pltpu.SemaphoreType.DMA,
            pltpu.SemaphoreType.DMA,
            pltpu.SemaphoreType.DMA,
        ],
        compiler_params=pltpu.CompilerParams(use_tc_tiling_on_sc=False,
                                             needs_layout_passes=False),
    )(_body)
    out5 = k(ids_t, embedding_weights)
    return out5.transpose(2, 4, 0, 1, 3).reshape(SEQ, POS, DIM)


# speedup vs baseline: 1.0186x; 1.0186x over previous
import functools

import jax
import jax.numpy as jnp
from jax import lax
from jax.experimental import pallas as pl
from jax.experimental.pallas import tpu as pltpu
from jax.experimental.pallas import tpu_sc as plsc

NUM_EMB = 1000000
DIM = 64
SEQ = 16384
POS = 50
NW = 32
BLK = 256
IC = SEQ // BLK
N_BLOCKS = POS * IC
BPW = N_BLOCKS // NW


def _body(ids_hbm, table_hbm, out_hbm, idx0, idx1, rows0, rows1,
          gsem0, gsem1, wsem0, wsem1):
    nc = 2
    wid = lax.axis_index("s") * nc + lax.axis_index("c")

    def fire(b, idx_v, rows_v, gsem):
        k = b // IC
        ic = b % IC
        pltpu.sync_copy(ids_hbm.at[k, pl.ds(ic * BLK, BLK)], idx_v)
        pltpu.async_copy(table_hbm.at[idx_v], rows_v, gsem)

    def drain_gather(rows_v, gsem):
        pltpu.make_async_copy(table_hbm.at[idx0], rows_v, gsem).wait()

    def write(b, rows_v, wsem):
        k = b // IC
        ic = b % IC
        pltpu.async_copy(rows_v, out_hbm.at[k, pl.ds(ic * BLK, BLK), :], wsem)

    def drain_write(rows_v, wsem):
        pltpu.make_async_copy(rows_v, out_hbm.at[0, pl.ds(0, BLK), :],
                              wsem).wait()

    base = wid * BPW
    fire(base, idx0, rows0, gsem0)

    def step(s, carry):
        b0 = base + 2 * s

        fire(b0 + 1, idx1, rows1, gsem1)
        drain_gather(rows0, gsem0)
        write(b0, rows0, wsem0)
        drain_write(rows0, wsem0)

        @pl.when(s < BPW // 2 - 1)
        def _():
            fire(b0 + 2, idx0, rows0, gsem0)
        drain_gather(rows1, gsem1)
        write(b0 + 1, rows1, wsem1)
        drain_write(rows1, wsem1)
        return carry

    lax.fori_loop(0, BPW // 2, step, 0)


def kernel(token_ids, embedding_weights):
    ids_t = token_ids.T.astype(jnp.int32)
    mesh = plsc.VectorSubcoreMesh(core_axis_name="c", subcore_axis_name="s")
    k = functools.partial(
        pl.kernel,
        mesh=mesh,
        out_type=jax.ShapeDtypeStruct((POS, SEQ, DIM), jnp.float32),
        scratch_types=[
            pltpu.VMEM((BLK,), jnp.int32),
            pltpu.VMEM((BLK,), jnp.int32),
            pltpu.VMEM((BLK, DIM), jnp.float32),
            pltpu.VMEM((BLK, DIM), jnp.float32),
            pltpu.SemaphoreType.DMA,
            pltpu.SemaphoreType.DMA,
            pltpu.SemaphoreType.DMA,
            pltpu.SemaphoreType.DMA,
        ],
        compiler_params=pltpu.CompilerParams(use_tc_tiling_on_sc=False,
                                             needs_layout_passes=False),
    )(_body)
    out_km = k(ids_t, embedding_weights)
    return out_km.transpose(1, 0, 2)
